# baseline (speedup 1.0000x reference)
import jax
import jax.numpy as jnp
from jax import lax
from jax.experimental import pallas as pl
from jax.experimental.pallas import tpu as pltpu
from jax.experimental.pallas import tpu_sc as plsc

_N = 100000
_D = 480
_DS = 288
_NC, _NS = 2, 16
_NW = _NC * _NS
_CB = 32
_NCB = _D // _CB
_CC = 256
_NCOLB = _N // _CC
_ROUNDS = _NCOLB // _NW
_NCOLB_SC = _ROUNDS * _NW
_TC_BLOCKS = _NCOLB - _NCOLB_SC + 1
_L = 16

_SC_SRC = tuple(
    cb * _CB if cb < 4 else (cb * _CB - 192 if cb >= 10 else None)
    for cb in range(_NCB)
)


def _combine_body(nf_hbm, dl_hbm, sc_hbm, out_hbm,
                  nf0, nf1, nf2, dl0, dl1, dl2, sc0, sc1, sc2,
                  is0, is1, is2, os0, os1, os2):
    nf_s = (nf0, nf1, nf2)
    dl_s = (dl0, dl1, dl2)
    sc_s = (sc0, sc1, sc2)
    isem = (is0, is1, is2)
    osem = (os0, os1, os2)

    wid = lax.axis_index("s") * _NC + lax.axis_index("c")

    def fire_in(cb, s, col0):
        pltpu.async_copy(nf_hbm.at[pl.ds(cb * _CB, _CB), pl.ds(col0, _CC)],
                         nf_s[s], isem[s])
        pltpu.async_copy(dl_hbm.at[pl.ds(cb * _CB, _CB), pl.ds(col0, _CC)],
                         dl_s[s], isem[s])
        if _SC_SRC[cb] is not None:
            pltpu.async_copy(
                sc_hbm.at[pl.ds(_SC_SRC[cb], _CB), pl.ds(col0, _CC)],
                sc_s[s], isem[s])

    def wait_in(cb, s):
        pltpu.make_async_copy(nf_hbm.at[pl.ds(0, _CB), pl.ds(0, _CC)],
                              nf_s[s], isem[s]).wait()
        pltpu.make_async_copy(dl_hbm.at[pl.ds(0, _CB), pl.ds(0, _CC)],
                              dl_s[s], isem[s]).wait()
        if _SC_SRC[cb] is not None:
            pltpu.make_async_copy(sc_hbm.at[pl.ds(0, _CB), pl.ds(0, _CC)],
                                  sc_s[s], isem[s]).wait()

    def fire_out(cb, s, col0):
        pltpu.async_copy(nf_s[s],
                         out_hbm.at[pl.ds(cb * _CB, _CB), pl.ds(col0, _CC)],
                         osem[s])

    def wait_out(s):
        pltpu.make_async_copy(nf_s[s],
                              out_hbm.at[pl.ds(0, _CB), pl.ds(0, _CC)],
                              osem[s]).wait()

    def compute(cb, s):
        nf_v, dl_v, sc_v = nf_s[s], dl_s[s], sc_s[s]
        has_sc = _SC_SRC[cb] is not None

        @plsc.parallel_loop(0, _CB, step=1, unroll=4)
        def row(r):
            for j in range(_CC // _L):
                plsc.addupdate(nf_v.at[r, pl.ds(j * _L, _L)],
                               dl_v[r, pl.ds(j * _L, _L)])
            if has_sc:
                for j in range(_CC // _L):
                    plsc.addupdate(nf_v.at[r, pl.ds(j * _L, _L)],
                                   sc_v[r, pl.ds(j * _L, _L)])

    col00 = wid * _CC
    fire_in(0, 0, col00)
    fire_in(1, 1, col00)

    @pl.loop(0, _ROUNDS)
    def _round(k):
        colb = k * _NW + wid
        col0 = colb * _CC
        for cb in range(_NCB):
            s = cb % 3
            wait_in(cb, s)
            compute(cb, s)
            fire_out(cb, s, col0)
            if cb == 0:
                @pl.when(k > 0)
                def _w():
                    wait_out(2)
            else:
                wait_out((cb + 2) % 3)
            if cb <= _NCB - 3:
                fire_in(cb + 2, (cb + 2) % 3, col0)
            else:
                cb2 = cb - (_NCB - 2)
                colb2 = colb + _NW

                @pl.when(colb2 < _NCOLB_SC)
                def _f():
                    fire_in(cb2, cb2 % 3, colb2 * _CC)

    wait_out(2)


def _tail_tc_body(prev_ref, nf_ref, dl_ref, sc_ref, out_ref):
    del prev_ref
    a = nf_ref[...] + dl_ref[...]
    s = sc_ref[...]
    out_ref[0:128, :] = a[0:128, :] + s[0:128, :]
    out_ref[128:320, :] = a[128:320, :]
    out_ref[320:480, :] = a[320:480, :] + s[128:288, :]


@jax.jit
def kernel(node_features, node_features_delta, node_scalars):
    nf_t = jnp.swapaxes(node_features, 0, 1)
    dl_t = jnp.swapaxes(node_features_delta, 0, 1)
    sc_t = jnp.swapaxes(node_scalars, 0, 1)
    run = pl.kernel(
        _combine_body,
        out_type=jax.ShapeDtypeStruct((_D, _N), jnp.float32),
        mesh=plsc.VectorSubcoreMesh(core_axis_name="c", subcore_axis_name="s",
                                    num_cores=_NC, num_subcores=_NS),
        scratch_types=[
            pltpu.VMEM((_CB, _CC), jnp.float32),
            pltpu.VMEM((_CB, _CC), jnp.float32),
            pltpu.VMEM((_CB, _CC), jnp.float32),
            pltpu.VMEM((_CB, _CC), jnp.float32),
            pltpu.VMEM((_CB, _CC), jnp.float32),
            pltpu.VMEM((_CB, _CC), jnp.float32),
            pltpu.VMEM((_CB, _CC), jnp.float32),
            pltpu.VMEM((_CB, _CC), jnp.float32),
            pltpu.VMEM((_CB, _CC), jnp.float32),
            pltpu.SemaphoreType.DMA,
            pltpu.SemaphoreType.DMA,
            pltpu.SemaphoreType.DMA,
            pltpu.SemaphoreType.DMA,
            pltpu.SemaphoreType.DMA,
            pltpu.SemaphoreType.DMA,
        ],
    )
    out_t = run(nf_t, dl_t, sc_t)

    out_t = pl.pallas_call(
        _tail_tc_body,
        grid=(_TC_BLOCKS,),
        in_specs=[
            pl.BlockSpec((_D, _CC), lambda i: (0, _NCOLB_SC + i)),
            pl.BlockSpec((_D, _CC), lambda i: (0, _NCOLB_SC + i)),
            pl.BlockSpec((_D, _CC), lambda i: (0, _NCOLB_SC + i)),
            pl.BlockSpec((_DS, _CC), lambda i: (0, _NCOLB_SC + i)),
        ],
        out_specs=pl.BlockSpec((_D, _CC), lambda i: (0, _NCOLB_SC + i)),
        out_shape=jax.ShapeDtypeStruct((_D, _N), jnp.float32),
        input_output_aliases={0: 0},
    )(out_t, nf_t, dl_t, sc_t)
    return jnp.swapaxes(out_t, 0, 1)


# speedup vs baseline: 1.1863x; 1.1863x over previous
import jax
import jax.numpy as jnp
from jax import lax
from jax.experimental import pallas as pl
from jax.experimental.pallas import tpu as pltpu
from jax.experimental.pallas import tpu_sc as plsc

_N = 100000
_D = 480
_DS = 288
_NC, _NS = 2, 16
_NW = _NC * _NS
_CB = 32
_NCB = _D // _CB
_CC = 256
_NCOLB = _N // _CC
_ROUNDS = _NCOLB // _NW
_NCOLB_SC = _ROUNDS * _NW
_TC_BLOCKS = _NCOLB - _NCOLB_SC + 1
_L = 16

_SC_SRC = tuple(
    cb * _CB if cb < 4 else (cb * _CB - 192 if cb >= 10 else None)
    for cb in range(_NCB)
)


def _combine_body(nf_hbm, dl_hbm, sc_hbm, out_hbm,
                  nf0, nf1, nf2, nf3, nf4, dl0, dl1, dl2, dl3, dl4,
                  sc0, sc1, sc2, sc3, sc4,
                  is0, is1, is2, is3, is4, os0, os1, os2, os3, os4):
    nf_s = (nf0, nf1, nf2, nf3, nf4)
    dl_s = (dl0, dl1, dl2, dl3, dl4)
    sc_s = (sc0, sc1, sc2, sc3, sc4)
    isem = (is0, is1, is2, is3, is4)
    osem = (os0, os1, os2, os3, os4)

    wid = lax.axis_index("s") * _NC + lax.axis_index("c")

    def fire_in(cb, s, col0):
        pltpu.async_copy(nf_hbm.at[pl.ds(cb * _CB, _CB), pl.ds(col0, _CC)],
                         nf_s[s], isem[s])
        pltpu.async_copy(dl_hbm.at[pl.ds(cb * _CB, _CB), pl.ds(col0, _CC)],
                         dl_s[s], isem[s])
        if _SC_SRC[cb] is not None:
            pltpu.async_copy(
                sc_hbm.at[pl.ds(_SC_SRC[cb], _CB), pl.ds(col0, _CC)],
                sc_s[s], isem[s])

    def wait_in(cb, s):
        pltpu.make_async_copy(nf_hbm.at[pl.ds(0, _CB), pl.ds(0, _CC)],
                              nf_s[s], isem[s]).wait()
        pltpu.make_async_copy(dl_hbm.at[pl.ds(0, _CB), pl.ds(0, _CC)],
                              dl_s[s], isem[s]).wait()
        if _SC_SRC[cb] is not None:
            pltpu.make_async_copy(sc_hbm.at[pl.ds(0, _CB), pl.ds(0, _CC)],
                                  sc_s[s], isem[s]).wait()

    def fire_out(cb, s, col0):
        pltpu.async_copy(nf_s[s],
                         out_hbm.at[pl.ds(cb * _CB, _CB), pl.ds(col0, _CC)],
                         osem[s])

    def wait_out(s):
        pltpu.make_async_copy(nf_s[s],
                              out_hbm.at[pl.ds(0, _CB), pl.ds(0, _CC)],
                              osem[s]).wait()

    def compute(cb, s):
        nf_v, dl_v, sc_v = nf_s[s], dl_s[s], sc_s[s]
        has_sc = _SC_SRC[cb] is not None

        @plsc.parallel_loop(0, _CB, step=1, unroll=2)
        def row(r):
            for j in range(_CC // _L):
                plsc.addupdate(nf_v.at[r, pl.ds(j * _L, _L)],
                               dl_v[r, pl.ds(j * _L, _L)])
            if has_sc:
                for j in range(_CC // _L):
                    plsc.addupdate(nf_v.at[r, pl.ds(j * _L, _L)],
                                   sc_v[r, pl.ds(j * _L, _L)])

    col00 = wid * _CC
    fire_in(0, 0, col00)
    fire_in(1, 1, col00)
    fire_in(2, 2, col00)

    @pl.loop(0, _ROUNDS)
    def _round(k):
        colb = k * _NW + wid
        col0 = colb * _CC
        for cb in range(_NCB):
            s = cb % 5
            wait_in(cb, s)
            compute(cb, s)
            fire_out(cb, s, col0)
            if cb == 0:
                @pl.when(k > 0)
                def _w():
                    wait_out(4)
            else:
                wait_out((cb + 4) % 5)
            if cb <= _NCB - 4:
                fire_in(cb + 3, (cb + 3) % 5, col0)
            else:
                cb2 = cb - (_NCB - 3)
                colb2 = colb + _NW

                @pl.when(colb2 < _NCOLB_SC)
                def _f():
                    fire_in(cb2, cb2 % 5, colb2 * _CC)

    wait_out(4)


def _tail_tc_body(prev_ref, nf_ref, dl_ref, sc_ref, out_ref):
    del prev_ref
    a = nf_ref[...] + dl_ref[...]
    s = sc_ref[...]
    out_ref[0:128, :] = a[0:128, :] + s[0:128, :]
    out_ref[128:320, :] = a[128:320, :]
    out_ref[320:480, :] = a[320:480, :] + s[128:288, :]


@jax.jit
def kernel(node_features, node_features_delta, node_scalars):
    nf_t = jnp.swapaxes(node_features, 0, 1)
    dl_t = jnp.swapaxes(node_features_delta, 0, 1)
    sc_t = jnp.swapaxes(node_scalars, 0, 1)
    run = pl.kernel(
        _combine_body,
        out_type=jax.ShapeDtypeStruct((_D, _N), jnp.float32),
        mesh=plsc.VectorSubcoreMesh(core_axis_name="c", subcore_axis_name="s",
                                    num_cores=_NC, num_subcores=_NS),
        scratch_types=(
            [pltpu.VMEM((_CB, _CC), jnp.float32)] * 15
            + [pltpu.SemaphoreType.DMA] * 10
        ),
    )
    out_t = run(nf_t, dl_t, sc_t)

    out_t = pl.pallas_call(
        _tail_tc_body,
        grid=(_TC_BLOCKS,),
        in_specs=[
            pl.BlockSpec((_D, _CC), lambda i: (0, _NCOLB_SC + i)),
            pl.BlockSpec((_D, _CC), lambda i: (0, _NCOLB_SC + i)),
            pl.BlockSpec((_D, _CC), lambda i: (0, _NCOLB_SC + i)),
            pl.BlockSpec((_DS, _CC), lambda i: (0, _NCOLB_SC + i)),
        ],
        out_specs=pl.BlockSpec((_D, _CC), lambda i: (0, _NCOLB_SC + i)),
        out_shape=jax.ShapeDtypeStruct((_D, _N), jnp.float32),
        input_output_aliases={0: 0},
    )(out_t, nf_t, dl_t, sc_t)
    return jnp.swapaxes(out_t, 0, 1)
